# Initial kernel scaffold; baseline (speedup 1.0000x reference)
import functools

import jax
import jax.numpy as jnp
from jax.experimental import pallas as pl


N = 10000
IN_DIM = 128
HID = 512
OUT_DIM = 30
ROW_BLK = 1000


def _attn_body(f_ref, w1_ref, att2_ref, out_ref):
    v = jnp.dot(w1_ref[...], att2_ref[...], preferred_element_type=jnp.float32)
    out_ref[...] = jnp.dot(f_ref[...], v, preferred_element_type=jnp.float32)


def _attn_scores(features, W1, att2):
    return pl.pallas_call(
        _attn_body,
        grid=(N // ROW_BLK,),
        in_specs=[
            pl.BlockSpec((ROW_BLK, IN_DIM), lambda i: (i, 0)),
            pl.BlockSpec((IN_DIM, HID), lambda i: (0, 0)),
            pl.BlockSpec((HID, 2), lambda i: (0, 0)),
        ],
        out_specs=pl.BlockSpec((ROW_BLK, 2), lambda i: (i, 0)),
        out_shape=jax.ShapeDtypeStruct((N, 2), jnp.float32),
    )(features, W1, att2)


def _mid_body(pf_ref, s_ref, w1_ref, w2_ref, out_ref):
    x = pf_ref[...] / (s_ref[...] + 1e-16)
    h1 = jnp.dot(x, w1_ref[...], preferred_element_type=jnp.float32)
    h1 = jnp.where(h1 > 0, h1, jnp.expm1(h1))
    out_ref[...] = jnp.dot(h1, w2_ref[...], preferred_element_type=jnp.float32)


def _mid_stage(pf, s, W1, W2):
    return pl.pallas_call(
        _mid_body,
        grid=(N // ROW_BLK,),
        in_specs=[
            pl.BlockSpec((ROW_BLK, IN_DIM), lambda i: (i, 0)),
            pl.BlockSpec((ROW_BLK, 1), lambda i: (i, 0)),
            pl.BlockSpec((IN_DIM, HID), lambda i: (0, 0)),
            pl.BlockSpec((HID, OUT_DIM), lambda i: (0, 0)),
        ],
        out_specs=pl.BlockSpec((ROW_BLK, OUT_DIM), lambda i: (i, 0)),
        out_shape=jax.ShapeDtypeStruct((N, OUT_DIM), jnp.float32),
    )(pf, s, W1, W2)


def _final_body(ph_ref, s_ref, w2t_ref, w1t_ref, out_ref):
    x = ph_ref[...] / (s_ref[...] + 1e-16)
    h3 = jnp.dot(x, w2t_ref[...], preferred_element_type=jnp.float32)
    h3 = jnp.where(h3 > 0, h3, jnp.expm1(h3))
    out_ref[...] = jnp.dot(h3, w1t_ref[...], preferred_element_type=jnp.float32)


def _final_stage(ph2, s, W2T, W1T):
    return pl.pallas_call(
        _final_body,
        grid=(N // ROW_BLK,),
        in_specs=[
            pl.BlockSpec((ROW_BLK, OUT_DIM), lambda i: (i, 0)),
            pl.BlockSpec((ROW_BLK, 1), lambda i: (i, 0)),
            pl.BlockSpec((OUT_DIM, HID), lambda i: (0, 0)),
            pl.BlockSpec((HID, IN_DIM), lambda i: (0, 0)),
        ],
        out_specs=pl.BlockSpec((ROW_BLK, IN_DIM), lambda i: (i, 0)),
        out_shape=jax.ShapeDtypeStruct((N, IN_DIM), jnp.float32),
    )(ph2, s, W2T, W1T)


def kernel(features, edge_index, W1, att_src1, att_dst1, W2):
    n = features.shape[0]
    src, dst = edge_index[0], edge_index[1]
    att2 = jnp.stack([att_src1, att_dst1], axis=1)

    a = _attn_scores(features, W1, att2)
    a_src, a_dst = a[:, 0], a[:, 1]

    e = jnp.exp(jax.nn.sigmoid(a_src[src] + a_dst[dst]))
    s = jax.ops.segment_sum(e, dst, num_segments=n)[:, None]
    pf = jax.ops.segment_sum(features[src] * e[:, None], dst, num_segments=n)

    h2 = _mid_stage(pf, s, W1, W2)

    ph2 = jax.ops.segment_sum(h2[src] * e[:, None], dst, num_segments=n)
    h4 = _final_stage(ph2, s, W2.T, W1.T)
    return (h2, h4)


# speedup vs baseline: 1.8632x; 1.8632x over previous
import functools

import jax
import jax.numpy as jnp
from jax.experimental import pallas as pl


N = 10000
IN_DIM = 128
HID = 512
OUT_DIM = 30
ROW_BLK = 1000


def _attn_body(f_ref, w1_ref, att2_ref, out_ref):
    v = jnp.dot(w1_ref[...], att2_ref[...], preferred_element_type=jnp.float32)
    out_ref[...] = jnp.dot(f_ref[...], v, preferred_element_type=jnp.float32)


def _attn_scores(features, W1, att2):
    return pl.pallas_call(
        _attn_body,
        grid=(N // ROW_BLK,),
        in_specs=[
            pl.BlockSpec((ROW_BLK, IN_DIM), lambda i: (i, 0)),
            pl.BlockSpec((IN_DIM, HID), lambda i: (0, 0)),
            pl.BlockSpec((HID, 2), lambda i: (0, 0)),
        ],
        out_specs=pl.BlockSpec((ROW_BLK, 2), lambda i: (i, 0)),
        out_shape=jax.ShapeDtypeStruct((N, 2), jnp.float32),
    )(features, W1, att2)


def _mid_body(pf_ref, s_ref, w1_ref, w2_ref, out_ref):
    x = pf_ref[...] / (s_ref[...] + 1e-16)
    h1 = jnp.dot(x, w1_ref[...], preferred_element_type=jnp.float32)
    h1 = jnp.where(h1 > 0, h1, jnp.exp(jnp.minimum(h1, 0.0)) - 1.0)
    out_ref[...] = jnp.dot(h1, w2_ref[...], preferred_element_type=jnp.float32)


def _mid_stage(pf, s, W1, W2):
    return pl.pallas_call(
        _mid_body,
        grid=(N // ROW_BLK,),
        in_specs=[
            pl.BlockSpec((ROW_BLK, IN_DIM), lambda i: (i, 0)),
            pl.BlockSpec((ROW_BLK, 1), lambda i: (i, 0)),
            pl.BlockSpec((IN_DIM, HID), lambda i: (0, 0)),
            pl.BlockSpec((HID, OUT_DIM), lambda i: (0, 0)),
        ],
        out_specs=pl.BlockSpec((ROW_BLK, OUT_DIM), lambda i: (i, 0)),
        out_shape=jax.ShapeDtypeStruct((N, OUT_DIM), jnp.float32),
    )(pf, s, W1, W2)


def _final_body(ph_ref, s_ref, w2t_ref, w1t_ref, out_ref):
    x = ph_ref[...] / (s_ref[...] + 1e-16)
    h3 = jnp.dot(x, w2t_ref[...], preferred_element_type=jnp.float32)
    h3 = jnp.where(h3 > 0, h3, jnp.exp(jnp.minimum(h3, 0.0)) - 1.0)
    out_ref[...] = jnp.dot(h3, w1t_ref[...], preferred_element_type=jnp.float32)


def _final_stage(ph2, s, W2T, W1T):
    return pl.pallas_call(
        _final_body,
        grid=(N // ROW_BLK,),
        in_specs=[
            pl.BlockSpec((ROW_BLK, OUT_DIM), lambda i: (i, 0)),
            pl.BlockSpec((ROW_BLK, 1), lambda i: (i, 0)),
            pl.BlockSpec((OUT_DIM, HID), lambda i: (0, 0)),
            pl.BlockSpec((HID, IN_DIM), lambda i: (0, 0)),
        ],
        out_specs=pl.BlockSpec((ROW_BLK, IN_DIM), lambda i: (i, 0)),
        out_shape=jax.ShapeDtypeStruct((N, IN_DIM), jnp.float32),
    )(ph2, s, W2T, W1T)


def kernel(features, edge_index, W1, att_src1, att_dst1, W2):
    n = features.shape[0]
    src, dst = edge_index[0], edge_index[1]
    att2 = jnp.stack([att_src1, att_dst1], axis=1)

    a = _attn_scores(features, W1, att2)
    a_src, a_dst = a[:, 0], a[:, 1]

    e = jnp.exp(jax.nn.sigmoid(a_src[src] + a_dst[dst]))
    s = jax.ops.segment_sum(e, dst, num_segments=n)[:, None]
    pf = jax.ops.segment_sum(features[src] * e[:, None], dst, num_segments=n)

    h2 = _mid_stage(pf, s, W1, W2)

    ph2 = jax.ops.segment_sum(h2[src] * e[:, None], dst, num_segments=n)
    h4 = _final_stage(ph2, s, W2.T, W1.T)
    return (h2, h4)


# speedup vs baseline: 11.7431x; 6.3028x over previous
import functools

import jax
import jax.numpy as jnp
from jax import lax
from jax.experimental import pallas as pl
from jax.experimental.pallas import tpu as pltpu
from jax.experimental.pallas import tpu_sc as plsc


N = 10000
E = 160000
IN_DIM = 128
HID = 512
OUT_DIM = 30

NC = 2
NS = 16
LANES = 16

NPAD = 10240
ROWS_PER_TILE = NPAD // NS
C = 128
W = NC * NS
CHUNKS = 40
EPAD = W * CHUNKS * C
IDX_ROWS = CHUNKS * C // 128
D1 = IN_DIM + 16
D2 = 32

ROW_BLK = 1024


def _attn_body(f_ref, w1_ref, att2_ref, out_ref):
    v = jnp.dot(w1_ref[...], att2_ref[...], preferred_element_type=jnp.float32)
    out_ref[...] = jnp.dot(f_ref[...], v, preferred_element_type=jnp.float32)


def _attn_scores(features, W1, att2):
    return pl.pallas_call(
        _attn_body,
        grid=(10,),
        in_specs=[
            pl.BlockSpec((1000, IN_DIM), lambda i: (i, 0)),
            pl.BlockSpec((IN_DIM, HID), lambda i: (0, 0)),
            pl.BlockSpec((HID, 2), lambda i: (0, 0)),
        ],
        out_specs=pl.BlockSpec((1000, 2), lambda i: (i, 0)),
        out_shape=jax.ShapeDtypeStruct((N, 2), jnp.float32),
    )(features, W1, att2)


def _mid_body(p0_ref, p1_ref, w1_ref, w2_ref, out_ref):
    p0 = p0_ref[...]
    p1 = p1_ref[...]
    x = p0[:, :IN_DIM] + p1[:, :IN_DIM]
    s = p0[:, IN_DIM:IN_DIM + 1] + p1[:, IN_DIM:IN_DIM + 1]
    x = x / (s + 1e-16)
    h1 = jnp.dot(x, w1_ref[...], preferred_element_type=jnp.float32)
    h1 = jnp.where(h1 > 0, h1, jnp.exp(jnp.minimum(h1, 0.0)) - 1.0)
    out_ref[...] = jnp.dot(h1, w2_ref[...], preferred_element_type=jnp.float32)


def _mid_stage(pf0, pf1, W1, W2p):
    return pl.pallas_call(
        _mid_body,
        grid=(NPAD // ROW_BLK,),
        in_specs=[
            pl.BlockSpec((ROW_BLK, D1), lambda i: (i, 0)),
            pl.BlockSpec((ROW_BLK, D1), lambda i: (i, 0)),
            pl.BlockSpec((IN_DIM, HID), lambda i: (0, 0)),
            pl.BlockSpec((HID, D2), lambda i: (0, 0)),
        ],
        out_specs=pl.BlockSpec((ROW_BLK, D2), lambda i: (i, 0)),
        out_shape=jax.ShapeDtypeStruct((NPAD, D2), jnp.float32),
    )(pf0, pf1, W1, W2p)


def _final_body(p0_ref, p1_ref, s0_ref, s1_ref, w2t_ref, w1t_ref, out_ref):
    x = p0_ref[...] + p1_ref[...]
    s = s0_ref[...] + s1_ref[...]
    x = x / (s + 1e-16)
    h3 = jnp.dot(x, w2t_ref[...], preferred_element_type=jnp.float32)
    h3 = jnp.where(h3 > 0, h3, jnp.exp(jnp.minimum(h3, 0.0)) - 1.0)
    out_ref[...] = jnp.dot(h3, w1t_ref[...], preferred_element_type=jnp.float32)


def _final_stage(ph0, ph1, s0, s1, W2pT, W1T):
    return pl.pallas_call(
        _final_body,
        grid=(NPAD // ROW_BLK,),
        in_specs=[
            pl.BlockSpec((ROW_BLK, D2), lambda i: (i, 0)),
            pl.BlockSpec((ROW_BLK, D2), lambda i: (i, 0)),
            pl.BlockSpec((ROW_BLK, 1), lambda i: (i, 0)),
            pl.BlockSpec((ROW_BLK, 1), lambda i: (i, 0)),
            pl.BlockSpec((D2, HID), lambda i: (0, 0)),
            pl.BlockSpec((HID, IN_DIM), lambda i: (0, 0)),
        ],
        out_specs=pl.BlockSpec((ROW_BLK, IN_DIM), lambda i: (i, 0)),
        out_shape=jax.ShapeDtypeStruct((NPAD, IN_DIM), jnp.float32),
    )(ph0, ph1, s0, s1, W2pT, W1T)


_MESH = plsc.VectorSubcoreMesh(
    core_axis_name="c", subcore_axis_name="s", num_cores=NC, num_subcores=NS)
_SC_PARAMS = pltpu.CompilerParams(
    needs_layout_passes=False, use_tc_tiling_on_sc=False)


def _zero_rows(zrows_ref, nrows, width):
    zero = jnp.zeros((LANES,), jnp.float32)
    for r in range(nrows):
        for d in range(width // LANES):
            zrows_ref[r, pl.ds(d * LANES, LANES)] = zero


def _scale_rows(rows_ref, e_ref, nslices):
    def body(j, _):
        jb = jnp.full((LANES,), j, dtype=jnp.int32)
        wv = plsc.load_gather(e_ref, [jb])
        for d in range(nslices):
            sl = rows_ref[j, pl.ds(d * LANES, LANES)]
            rows_ref[j, pl.ds(d * LANES, LANES)] = sl * wv
        return ()

    lax.fori_loop(0, C, body, ())


def _copy_out(sh_ref, out_ref, cid, sid):
    r0 = sid * ROWS_PER_TILE

    @pl.when(cid == 0)
    def _():
        pltpu.sync_copy(sh_ref.at[pl.ds(r0, ROWS_PER_TILE)],
                        out_ref.at[0, pl.ds(r0, ROWS_PER_TILE)])

    @pl.when(cid == 1)
    def _():
        pltpu.sync_copy(sh_ref.at[pl.ds(r0, ROWS_PER_TILE)],
                        out_ref.at[1, pl.ds(r0, ROWS_PER_TILE)])


def _w_body(src2d, dst2d, asrc_h, adst_h,
            e_out,
            asrc_v, adst_v, srcv, dstv, e_v):
    cid = lax.axis_index("c")
    sid = lax.axis_index("s")
    wid = cid * NS + sid

    pltpu.sync_copy(asrc_h, asrc_v)
    pltpu.sync_copy(adst_h, adst_v)
    pltpu.sync_copy(src2d.at[pl.ds(wid * IDX_ROWS, IDX_ROWS)], srcv)
    pltpu.sync_copy(dst2d.at[pl.ds(wid * IDX_ROWS, IDX_ROWS)], dstv)

    eb = wid * IDX_ROWS * 128

    def row_iter(r, _):
        for i in range(128 // LANES):
            sl = srcv[r, pl.ds(i * LANES, LANES)]
            dl = dstv[r, pl.ds(i * LANES, LANES)]
            av = plsc.load_gather(asrc_v, [sl])
            bv = plsc.load_gather(adst_v, [dl])
            sig = 1.0 / (1.0 + jnp.exp(-(av + bv)))
            ev = jnp.exp(sig)
            ids = eb + r * 128 + i * LANES + lax.iota(jnp.int32, LANES)
            ev = jnp.where(ids < E, ev, 0.0)
            e_v[pl.ds(r * 128 + i * LANES, LANES)] = ev
        return ()

    lax.fori_loop(0, IDX_ROWS, row_iter, ())
    pltpu.sync_copy(e_v, e_out.at[pl.ds(eb, IDX_ROWS * 128)])


def _sc_weights(src2d, dst2d, a_src, a_dst):
    f = functools.partial(
        pl.kernel,
        out_type=jax.ShapeDtypeStruct((EPAD,), jnp.float32),
        mesh=_MESH,
        compiler_params=_SC_PARAMS,
        scratch_types=[
            pltpu.VMEM((N,), jnp.float32),
            pltpu.VMEM((N,), jnp.float32),
            pltpu.VMEM((IDX_ROWS, 128), jnp.int32),
            pltpu.VMEM((IDX_ROWS, 128), jnp.int32),
            pltpu.VMEM((IDX_ROWS * 128,), jnp.float32),
        ],
    )(_w_body)
    return f(src2d, dst2d, a_src, a_dst)


def _make_prop_body(D):
    def body(src2d, dst2d, e_h, tab_h,
             out_h,
             srcc, dstc, e_c, rows_v, zrows, acc_sh, gsem):
        cid = lax.axis_index("c")
        sid = lax.axis_index("s")
        wid = cid * NS + sid

        _zero_rows(zrows, 64, D)
        for k in range(ROWS_PER_TILE // 64):
            pltpu.sync_copy(
                zrows, acc_sh.at[pl.ds(sid * ROWS_PER_TILE + k * 64, 64)])
        plsc.subcore_barrier()

        def chunk(k, _):
            row = wid * IDX_ROWS + k
            cb = row * C
            pltpu.sync_copy(src2d.at[pl.ds(row, 1)], srcc)
            pltpu.sync_copy(dst2d.at[pl.ds(row, 1)], dstc)
            g = pltpu.async_copy(tab_h.at[srcc.at[0]], rows_v, gsem)
            pltpu.sync_copy(e_h.at[pl.ds(cb, C)], e_c)
            g.wait()
            _scale_rows(rows_v, e_c, D // LANES)
            pltpu.sync_copy(rows_v, acc_sh.at[dstc.at[0]], add=True)
            return ()

        lax.fori_loop(0, CHUNKS, chunk, ())

        plsc.subcore_barrier()
        _copy_out(acc_sh, out_h, cid, sid)

    return body


def _sc_propagate(src2d, dst2d, e, tab, D):
    f = functools.partial(
        pl.kernel,
        out_type=jax.ShapeDtypeStruct((NC, NPAD, D), jnp.float32),
        mesh=_MESH,
        compiler_params=_SC_PARAMS,
        scratch_types=[
            pltpu.VMEM((1, 128), jnp.int32),
            pltpu.VMEM((1, 128), jnp.int32),
            pltpu.VMEM((C,), jnp.float32),
            pltpu.VMEM((C, D), jnp.float32),
            pltpu.VMEM((64, D), jnp.float32),
            pltpu.VMEM_SHARED((NPAD, D), jnp.float32),
            pltpu.SemaphoreType.DMA,
        ],
    )(_make_prop_body(D))
    return f(src2d, dst2d, e, tab)


def kernel(features, edge_index, W1, att_src1, att_dst1, W2):
    src = edge_index[0]
    dst = edge_index[1]
    src2d = jnp.pad(src, (0, EPAD - E)).reshape(EPAD // 128, 128)
    dst2d = jnp.pad(dst, (0, EPAD - E)).reshape(EPAD // 128, 128)

    att2 = jnp.stack([att_src1, att_dst1], axis=1)
    a = _attn_scores(features, W1, att2)
    a_src = a[:, 0] + 0.0
    a_dst = a[:, 1] + 0.0

    fx = jnp.concatenate(
        [features, jnp.ones((N, 1), jnp.float32),
         jnp.zeros((N, D1 - IN_DIM - 1), jnp.float32)], axis=1)

    e = _sc_weights(src2d, dst2d, a_src, a_dst)
    pf_part = _sc_propagate(src2d, dst2d, e, fx, D1)

    W2p = jnp.pad(W2, ((0, 0), (0, D2 - OUT_DIM)))
    h2pad = _mid_stage(pf_part[0], pf_part[1], W1, W2p)

    ph_part = _sc_propagate(src2d, dst2d, e, h2pad, D2)

    s0 = pf_part[0, :, IN_DIM:IN_DIM + 1]
    s1 = pf_part[1, :, IN_DIM:IN_DIM + 1]
    h4 = _final_stage(ph_part[0], ph_part[1], s0, s1, W2p.T, W1.T)

    return (h2pad[:N, :OUT_DIM], h4[:N])


# speedup vs baseline: 16.7471x; 1.4261x over previous
import functools

import jax
import jax.numpy as jnp
from jax import lax
from jax.experimental import pallas as pl
from jax.experimental.pallas import tpu as pltpu
from jax.experimental.pallas import tpu_sc as plsc


N = 10000
E = 160000
IN_DIM = 128
HID = 512
OUT_DIM = 30

NC = 2
NS = 16
LANES = 16

NPAD = 10240
ROWS_PER_TILE = NPAD // NS
W = NC * NS
EPW = 5120
EPAD = W * EPW
IDX_ROWS = EPW // 128
D1 = IN_DIM
D2 = 32

ROW_BLK = 1024


def _attn_body(f_ref, w1_ref, att2_ref, out_ref):
    v = jnp.dot(w1_ref[...], att2_ref[...], preferred_element_type=jnp.float32)
    out_ref[...] = jnp.dot(f_ref[...], v, preferred_element_type=jnp.float32)


def _attn_scores(features, W1, att2):
    return pl.pallas_call(
        _attn_body,
        grid=(10,),
        in_specs=[
            pl.BlockSpec((1000, IN_DIM), lambda i: (i, 0)),
            pl.BlockSpec((IN_DIM, HID), lambda i: (0, 0)),
            pl.BlockSpec((HID, 2), lambda i: (0, 0)),
        ],
        out_specs=pl.BlockSpec((1000, 2), lambda i: (i, 0)),
        out_shape=jax.ShapeDtypeStruct((N, 2), jnp.float32),
    )(features, W1, att2)


def _mid_body(p0_ref, p1_ref, s0_ref, s1_ref, w1_ref, w2_ref, out_ref):
    x = p0_ref[...] + p1_ref[...]
    s = s0_ref[...] + s1_ref[...]
    x = x / (s + 1e-16)
    h1 = jnp.dot(x, w1_ref[...], preferred_element_type=jnp.float32)
    h1 = jnp.where(h1 > 0, h1, jnp.exp(jnp.minimum(h1, 0.0)) - 1.0)
    out_ref[...] = jnp.dot(h1, w2_ref[...], preferred_element_type=jnp.float32)


def _mid_stage(pf0, pf1, s0, s1, W1, W2p):
    return pl.pallas_call(
        _mid_body,
        grid=(NPAD // ROW_BLK,),
        in_specs=[
            pl.BlockSpec((ROW_BLK, D1), lambda i: (i, 0)),
            pl.BlockSpec((ROW_BLK, D1), lambda i: (i, 0)),
            pl.BlockSpec((ROW_BLK, 1), lambda i: (i, 0)),
            pl.BlockSpec((ROW_BLK, 1), lambda i: (i, 0)),
            pl.BlockSpec((IN_DIM, HID), lambda i: (0, 0)),
            pl.BlockSpec((HID, D2), lambda i: (0, 0)),
        ],
        out_specs=pl.BlockSpec((ROW_BLK, D2), lambda i: (i, 0)),
        out_shape=jax.ShapeDtypeStruct((NPAD, D2), jnp.float32),
    )(pf0, pf1, s0, s1, W1, W2p)


def _final_body(p0_ref, p1_ref, s0_ref, s1_ref, w2t_ref, w1t_ref, out_ref):
    x = p0_ref[...] + p1_ref[...]
    s = s0_ref[...] + s1_ref[...]
    x = x / (s + 1e-16)
    h3 = jnp.dot(x, w2t_ref[...], preferred_element_type=jnp.float32)
    h3 = jnp.where(h3 > 0, h3, jnp.exp(jnp.minimum(h3, 0.0)) - 1.0)
    out_ref[...] = jnp.dot(h3, w1t_ref[...], preferred_element_type=jnp.float32)


def _final_stage(ph0, ph1, s0, s1, W2pT, W1T):
    return pl.pallas_call(
        _final_body,
        grid=(NPAD // ROW_BLK,),
        in_specs=[
            pl.BlockSpec((ROW_BLK, D2), lambda i: (i, 0)),
            pl.BlockSpec((ROW_BLK, D2), lambda i: (i, 0)),
            pl.BlockSpec((ROW_BLK, 1), lambda i: (i, 0)),
            pl.BlockSpec((ROW_BLK, 1), lambda i: (i, 0)),
            pl.BlockSpec((D2, HID), lambda i: (0, 0)),
            pl.BlockSpec((HID, IN_DIM), lambda i: (0, 0)),
        ],
        out_specs=pl.BlockSpec((ROW_BLK, IN_DIM), lambda i: (i, 0)),
        out_shape=jax.ShapeDtypeStruct((NPAD, IN_DIM), jnp.float32),
    )(ph0, ph1, s0, s1, W2pT, W1T)


_MESH = plsc.VectorSubcoreMesh(
    core_axis_name="c", subcore_axis_name="s", num_cores=NC, num_subcores=NS)
_SC_PARAMS = pltpu.CompilerParams(
    needs_layout_passes=False, use_tc_tiling_on_sc=False)


def _zero_rows(zrows_ref, nrows, width):
    zero = jnp.zeros((LANES,), jnp.float32)
    for r in range(nrows):
        for d in range(width // LANES):
            zrows_ref[r, pl.ds(d * LANES, LANES)] = zero


def _scale_rows(rows_ref, e_ref, nslices, count, base):
    def body(j2, _):
        j = base + j2 * 2
        for u in range(2):
            jb = jnp.full((LANES,), j + u, dtype=jnp.int32)
            wv = plsc.load_gather(e_ref, [jb])
            for d in range(nslices):
                sl = rows_ref[j + u, pl.ds(d * LANES, LANES)]
                rows_ref[j + u, pl.ds(d * LANES, LANES)] = sl * wv
        return ()

    lax.fori_loop(0, count // 2, body, ())


def _copy_out(sh_ref, out_ref, cid, sid):
    r0 = sid * ROWS_PER_TILE

    @pl.when(cid == 0)
    def _():
        pltpu.sync_copy(sh_ref.at[pl.ds(r0, ROWS_PER_TILE)],
                        out_ref.at[0, pl.ds(r0, ROWS_PER_TILE)])

    @pl.when(cid == 1)
    def _():
        pltpu.sync_copy(sh_ref.at[pl.ds(r0, ROWS_PER_TILE)],
                        out_ref.at[1, pl.ds(r0, ROWS_PER_TILE)])


def _w_body(src2d, dst2d, asrc_h, adst_h,
            e_out, s_out,
            asrc_v, adst_v, srcv, dstv, e_v, zbuf, s_sh):
    cid = lax.axis_index("c")
    sid = lax.axis_index("s")
    wid = cid * NS + sid

    pltpu.sync_copy(asrc_h, asrc_v)
    pltpu.sync_copy(adst_h, adst_v)
    pltpu.sync_copy(src2d.at[pl.ds(wid * IDX_ROWS, IDX_ROWS)], srcv)
    pltpu.sync_copy(dst2d.at[pl.ds(wid * IDX_ROWS, IDX_ROWS)], dstv)

    zero = jnp.zeros((LANES,), jnp.float32)
    for i in range(ROWS_PER_TILE // LANES):
        zbuf[pl.ds(i * LANES, LANES)] = zero
    pltpu.sync_copy(zbuf, s_sh.at[pl.ds(sid * ROWS_PER_TILE, ROWS_PER_TILE)])
    plsc.subcore_barrier()

    eb = wid * EPW

    def row_iter(r, _):
        for i in range(128 // LANES):
            sl = srcv[r, pl.ds(i * LANES, LANES)]
            dl = dstv[r, pl.ds(i * LANES, LANES)]
            av = plsc.load_gather(asrc_v, [sl])
            bv = plsc.load_gather(adst_v, [dl])
            sig = 1.0 / (1.0 + jnp.exp(-(av + bv)))
            ev = jnp.exp(sig)
            ids = eb + r * 128 + i * LANES + lax.iota(jnp.int32, LANES)
            ev = jnp.where(ids < E, ev, 0.0)
            e_v[pl.ds(r * 128 + i * LANES, LANES)] = ev
        pltpu.sync_copy(e_v.at[pl.ds(r * 128, 128)],
                        s_sh.at[dstv.at[r]], add=True)
        return ()

    lax.fori_loop(0, IDX_ROWS, row_iter, ())
    pltpu.sync_copy(e_v, e_out.at[pl.ds(eb, EPW)])

    plsc.subcore_barrier()
    r0 = sid * ROWS_PER_TILE

    @pl.when(cid == 0)
    def _():
        pltpu.sync_copy(s_sh.at[pl.ds(r0, ROWS_PER_TILE)],
                        s_out.at[0, pl.ds(r0, ROWS_PER_TILE)])

    @pl.when(cid == 1)
    def _():
        pltpu.sync_copy(s_sh.at[pl.ds(r0, ROWS_PER_TILE)],
                        s_out.at[1, pl.ds(r0, ROWS_PER_TILE)])


def _sc_weights(src2d, dst2d, a_src, a_dst):
    f = functools.partial(
        pl.kernel,
        out_type=[
            jax.ShapeDtypeStruct((EPAD,), jnp.float32),
            jax.ShapeDtypeStruct((NC, NPAD), jnp.float32),
        ],
        mesh=_MESH,
        compiler_params=_SC_PARAMS,
        scratch_types=[
            pltpu.VMEM((N,), jnp.float32),
            pltpu.VMEM((N,), jnp.float32),
            pltpu.VMEM((IDX_ROWS, 128), jnp.int32),
            pltpu.VMEM((IDX_ROWS, 128), jnp.int32),
            pltpu.VMEM((EPW,), jnp.float32),
            pltpu.VMEM((ROWS_PER_TILE,), jnp.float32),
            pltpu.VMEM_SHARED((NPAD,), jnp.float32),
        ],
    )(_w_body)
    return f(src2d, dst2d, a_src, a_dst)


def _make_prop_body(D, C, CHUNKS):
    NB = C // 128

    def body(src2d, dst2d, e_h, tab_h,
             out_h,
             srcv, dstv, e_c, rows_v, zrows, acc_sh, gsem, ssem, esem):
        cid = lax.axis_index("c")
        sid = lax.axis_index("s")
        wid = cid * NS + sid

        _zero_rows(zrows, 64, D)
        for k in range(ROWS_PER_TILE // 64):
            pltpu.sync_copy(
                zrows, acc_sh.at[pl.ds(sid * ROWS_PER_TILE + k * 64, 64)])

        def stage_idx(k, slot):
            pltpu.sync_copy(src2d.at[pl.ds(wid * IDX_ROWS + k * NB, NB)],
                            srcv.at[pl.ds(slot * NB, NB)])
            pltpu.sync_copy(dst2d.at[pl.ds(wid * IDX_ROWS + k * NB, NB)],
                            dstv.at[pl.ds(slot * NB, NB)])

        def gather_start(slot, buf):
            for b in range(NB):
                pltpu.async_copy(
                    tab_h.at[srcv.at[slot * NB + b]],
                    rows_v.at[pl.ds((buf * NB + b) * 128, 128)], gsem)

        def gather_wait():
            for b in range(NB):
                pltpu.make_async_copy(
                    tab_h.at[srcv.at[0]],
                    rows_v.at[pl.ds(b * 128, 128)], gsem).wait()

        def e_start(k, buf):
            pltpu.async_copy(e_h.at[pl.ds((wid * CHUNKS + k) * C, C)],
                             e_c.at[pl.ds(buf * C, C)], esem)

        def e_wait():
            pltpu.make_async_copy(e_h.at[pl.ds(0, C)],
                                  e_c.at[pl.ds(0, C)], esem).wait()

        stage_idx(0, 0)
        stage_idx(1, 1)
        gather_start(0, 0)
        e_start(0, 0)
        gather_start(1, 1)
        e_start(1, 1)

        plsc.subcore_barrier()

        def chunk(k, _):
            buf = lax.rem(k, 2)
            slot = lax.rem(k, 4)
            gather_wait()
            e_wait()
            _scale_rows(rows_v, e_c, D // LANES, C, buf * C)
            descs = [
                pltpu.async_copy(
                    rows_v.at[pl.ds((buf * NB + b) * 128, 128)],
                    acc_sh.at[dstv.at[slot * NB + b]], ssem, add=True)
                for b in range(NB)
            ]
            for d in descs:
                d.wait()

            @pl.when(k + 2 < CHUNKS)
            def _():
                nslot = lax.rem(k + 2, 4)
                stage_idx(k + 2, nslot)
                gather_start(nslot, buf)
                e_start(k + 2, buf)

            return ()

        lax.fori_loop(0, CHUNKS, chunk, ())

        plsc.subcore_barrier()
        _copy_out(acc_sh, out_h, cid, sid)

    return body


def _sc_propagate(src2d, dst2d, e, tab, D, C):
    CHUNKS = EPW // C
    f = functools.partial(
        pl.kernel,
        out_type=jax.ShapeDtypeStruct((NC, NPAD, D), jnp.float32),
        mesh=_MESH,
        compiler_params=_SC_PARAMS,
        scratch_types=[
            pltpu.VMEM((4 * (C // 128), 128), jnp.int32),
            pltpu.VMEM((4 * (C // 128), 128), jnp.int32),
            pltpu.VMEM((2 * C,), jnp.float32),
            pltpu.VMEM((2 * C, D), jnp.float32),
            pltpu.VMEM((64, D), jnp.float32),
            pltpu.VMEM_SHARED((NPAD, D), jnp.float32),
            pltpu.SemaphoreType.DMA,
            pltpu.SemaphoreType.DMA,
            pltpu.SemaphoreType.DMA,
        ],
    )(_make_prop_body(D, C, CHUNKS))
    return f(src2d, dst2d, e, tab)


def kernel(features, edge_index, W1, att_src1, att_dst1, W2):
    src = edge_index[0]
    dst = edge_index[1]
    src2d = jnp.pad(src, (0, EPAD - E)).reshape(EPAD // 128, 128)
    dst2d = jnp.pad(dst, (0, EPAD - E)).reshape(EPAD // 128, 128)

    att2 = jnp.stack([att_src1, att_dst1], axis=1)
    a = _attn_scores(features, W1, att2)
    a_src = a[:, 0] + 0.0
    a_dst = a[:, 1] + 0.0

    e, s_part = _sc_weights(src2d, dst2d, a_src, a_dst)
    s0 = s_part[0].reshape(NPAD, 1)
    s1 = s_part[1].reshape(NPAD, 1)

    pf_part = _sc_propagate(src2d, dst2d, e, features, D1, 128)

    W2p = jnp.pad(W2, ((0, 0), (0, D2 - OUT_DIM)))
    h2pad = _mid_stage(pf_part[0], pf_part[1], s0, s1, W1, W2p)

    ph_part = _sc_propagate(src2d, dst2d, e, h2pad, D2, 512)

    h4 = _final_stage(ph_part[0], ph_part[1], s0, s1, W2p.T, W1.T)

    return (h2pad[:N, :OUT_DIM], h4[:N])
